# baseline (speedup 1.0000x reference)
import jax
import jax.numpy as jnp
from jax.experimental import pallas as pl


def _body(tokens_ref, months_ref, ch_ref, pos_ref, month_ref, out_ref):
    t = 12
    mrow = months_ref[0]
    sel = (jax.lax.broadcasted_iota(jnp.int32, (t, t), 0) == mrow)
    month_e = jax.lax.dot_general(
        sel.astype(jnp.float32), month_ref[...],
        dimension_numbers=(((0,), (0,)), ((), ())),
        preferred_element_type=jnp.float32)
    ch = jnp.broadcast_to(ch_ref[...][None], (t, 3, 192))
    pe = jnp.broadcast_to(pos_ref[:t][:, None], (t, 3, 192))
    me = jnp.broadcast_to(month_e[:, None], (t, 3, 192))
    zero = jnp.zeros((t, 3, 192), jnp.float32)
    addend = jnp.concatenate([ch, pe, me, zero], axis=-1)
    out_ref[...] = tokens_ref[...] + addend[None, None, :, :, None, :]


def kernel(tokens, timestamps, channel_embed, pos_embed, month_table):
    b, h, w, t, bs, d = tokens.shape
    n = d // 4
    months = timestamps[:, :, 1].astype(jnp.int32).reshape(b, 1, t)
    tok = jnp.transpose(tokens, (0, 1, 3, 4, 2, 5))
    hb = 2
    tok_spec = pl.BlockSpec((1, hb, t, bs, w, d), lambda i, j: (i, j, 0, 0, 0, 0))
    out = pl.pallas_call(
        _body,
        grid=(b, h // hb),
        in_specs=[
            tok_spec,
            pl.BlockSpec((1, 1, t), lambda i, j: (i, 0, 0)),
            pl.BlockSpec((bs, n), lambda i, j: (0, 0)),
            pl.BlockSpec((pos_embed.shape[0], n), lambda i, j: (0, 0)),
            pl.BlockSpec((t, n), lambda i, j: (0, 0)),
        ],
        out_specs=tok_spec,
        out_shape=jax.ShapeDtypeStruct((b, h, t, bs, w, d), tokens.dtype),
    )(tok, months, channel_embed, pos_embed, month_table)
    return jnp.transpose(out, (0, 1, 4, 2, 3, 5))


# speedup vs baseline: 1.0567x; 1.0567x over previous
import jax
import jax.numpy as jnp
from jax.experimental import pallas as pl


def _body(tokens_ref, months_ref, ch_ref, pos_ref, month_ref, out_ref):
    t = 12
    mrow = months_ref[0]
    sel = (jax.lax.broadcasted_iota(jnp.int32, (t, t), 0) == mrow)
    month_e = jax.lax.dot_general(
        sel.astype(jnp.float32), month_ref[...],
        dimension_numbers=(((0,), (0,)), ((), ())),
        preferred_element_type=jnp.float32)
    ch = jnp.broadcast_to(ch_ref[...][None], (t, 3, 192))
    pe = jnp.broadcast_to(pos_ref[:t][:, None], (t, 3, 192))
    me = jnp.broadcast_to(month_e[:, None], (t, 3, 192))
    zero = jnp.zeros((t, 3, 192), jnp.float32)
    addend = jnp.concatenate([ch, pe, me, zero], axis=-1)
    out_ref[...] = tokens_ref[...] + addend[None, None, :, :, None, :]


def kernel(tokens, timestamps, channel_embed, pos_embed, month_table):
    b, h, w, t, bs, d = tokens.shape
    n = d // 4
    months = timestamps[:, :, 1].astype(jnp.int32).reshape(b, 1, t)
    tok = jnp.transpose(tokens, (0, 1, 3, 4, 2, 5))
    hb = 8
    tok_spec = pl.BlockSpec((1, hb, t, bs, w, d), lambda i, j: (i, j, 0, 0, 0, 0))
    out = pl.pallas_call(
        _body,
        grid=(b, h // hb),
        in_specs=[
            tok_spec,
            pl.BlockSpec((1, 1, t), lambda i, j: (i, 0, 0)),
            pl.BlockSpec((bs, n), lambda i, j: (0, 0)),
            pl.BlockSpec((pos_embed.shape[0], n), lambda i, j: (0, 0)),
            pl.BlockSpec((t, n), lambda i, j: (0, 0)),
        ],
        out_specs=tok_spec,
        out_shape=jax.ShapeDtypeStruct((b, h, t, bs, w, d), tokens.dtype),
    )(tok, months, channel_embed, pos_embed, month_table)
    return jnp.transpose(out, (0, 1, 4, 2, 3, 5))
